# Initial kernel scaffold; baseline (speedup 1.0000x reference)
import jax
import jax.numpy as jnp
from jax.experimental import pallas as pl


def kernel(x, edge_index_connections, edge_index_destinations, W1l, W1r, b1, W2l, W2r, b2, W3l, W3r, b3, W4l, W4r, b4):
    raise NotImplementedError("write your pallas kernel here")


# speedup vs baseline: 1.0075x; 1.0075x over previous
import functools

import jax
import jax.numpy as jnp
from jax import lax
from jax.experimental import pallas as pl
from jax.experimental.pallas import tpu as pltpu

N = 10000
F_IN = 256
H = 512
M_BLK = 1000


def _sage_mm_body(agg_ref, inv_ref, h_ref, wl_ref, wr_ref, b_ref, out_ref):
    mean = agg_ref[...] * inv_ref[...]
    yl = lax.dot_general(mean, wl_ref[...], (((1,), (1,)), ((), ())),
                         preferred_element_type=jnp.float32)
    yr = lax.dot_general(h_ref[...], wr_ref[...], (((1,), (1,)), ((), ())),
                         preferred_element_type=jnp.float32)
    out_ref[...] = jnp.maximum(yl + yr + b_ref[...], 0.0)


@functools.partial(jax.jit, static_argnames=("f_in",))
def _sage_mm(agg, inv, h, wl, wr, b, f_in):
    grid = N // M_BLK
    return pl.pallas_call(
        _sage_mm_body,
        grid=(grid,),
        in_specs=[
            pl.BlockSpec((M_BLK, f_in), lambda i: (i, 0)),
            pl.BlockSpec((M_BLK, 1), lambda i: (i, 0)),
            pl.BlockSpec((M_BLK, f_in), lambda i: (i, 0)),
            pl.BlockSpec((H, f_in), lambda i: (0, 0)),
            pl.BlockSpec((H, f_in), lambda i: (0, 0)),
            pl.BlockSpec((1, H), lambda i: (0, 0)),
        ],
        out_specs=pl.BlockSpec((M_BLK, H), lambda i: (i, 0)),
        out_shape=jax.ShapeDtypeStruct((N, H), jnp.float32),
    )(agg, inv, h, wl, wr, b.reshape(1, H))


def _aggregate(h, src, dst):
    msgs = jnp.take(h, src, axis=0)
    agg = jax.ops.segment_sum(msgs, dst, num_segments=N)
    return agg


def _layer(h, src, dst, inv, wl, wr, b):
    agg = _aggregate(h, src, dst)
    return _sage_mm(agg, inv, h, wl, wr, b, h.shape[1])


def kernel(x, edge_index_connections, edge_index_destinations,
           W1l, W1r, b1, W2l, W2r, b2, W3l, W3r, b3, W4l, W4r, b4):
    src_c, dst_c = edge_index_connections[0], edge_index_connections[1]
    src_d, dst_d = edge_index_destinations[0], edge_index_destinations[1]
    ones = jnp.ones((src_c.shape[0],), jnp.float32)
    cnt_c = jax.ops.segment_sum(ones, dst_c, num_segments=N)
    cnt_d = jax.ops.segment_sum(ones, dst_d, num_segments=N)
    inv_c = (1.0 / jnp.clip(cnt_c, 1.0, None)).reshape(N, 1)
    inv_d = (1.0 / jnp.clip(cnt_d, 1.0, None)).reshape(N, 1)

    h = _layer(x, src_c, dst_c, inv_c, W1l, W1r, b1)
    h = _layer(h, src_c, dst_c, inv_c, W4l, W4r, b4)
    h = _layer(h, src_c, dst_c, inv_c, W4l, W4r, b4)
    h = _layer(h, src_d, dst_d, inv_d, W2l, W2r, b2)
    h = _layer(h, src_c, dst_c, inv_c, W3l, W3r, b3)
    h = _layer(h, src_c, dst_c, inv_c, W3l, W3r, b3)
    return h


# speedup vs baseline: 1.4095x; 1.3989x over previous
import functools

import jax
import jax.numpy as jnp
from jax import lax
from jax.experimental import pallas as pl
from jax.experimental.pallas import tpu as pltpu
from jax.experimental.pallas import tpu_sc as plsc

N = 10000
H = 512
E_EXTRA = 1024
NCH = 64
CH = 160
OUT_ROWS = 168
DUMMY = 164
K = 64
M_BLK = 1000


def _agg_body(F, h_hbm, src_hbm, dst_hbm, off_hbm, zeros_hbm, out_hbm,
              offv, raw_s, raw_d, srcm, rows, outbuf, sem):
    c = lax.axis_index("c")
    s = lax.axis_index("s")
    w = 2 * s + c
    iota = lax.iota(jnp.int32, 16)
    pltpu.sync_copy(off_hbm, offv)

    def chunk_body(k, carry):
        cid = 2 * w + k
        ov = offv[pl.ds(cid, 16)]
        o0 = ov[0]
        o1 = ov[1]
        astart = o0 & (-8)
        nb = lax.div(o1 - astart + (K - 1), K)

        pltpu.sync_copy(zeros_hbm, outbuf.at[pl.ds(0, CH)])

        def group(g, gcarry):
            b = pl.multiple_of(astart + lax.div(g, 4) * K, 8)

            @pl.when(lax.rem(g, 4) == 0)
            def _():
                pltpu.sync_copy(src_hbm.at[pl.ds(b, K)], raw_s)
                pltpu.sync_copy(dst_hbm.at[pl.ds(b, K)], raw_d)
                for j in range(K // 16):
                    e = b + j * 16 + iota
                    valid = (e >= o0) & (e < o1)
                    sv = raw_s[pl.ds(j * 16, 16)]
                    srcm[pl.ds(j * 16, 16)] = jnp.where(valid, sv, 0)
                pltpu.async_copy(h_hbm.at[srcm], rows, sem).wait()

            j = lax.rem(g, 4)
            e = b + j * 16 + iota
            valid = (e >= o0) & (e < o1)
            dv = raw_d[pl.ds(j * 16, 16)]
            dm = jnp.where(valid, dv, DUMMY)
            for l in range(16):
                d = dm[l]
                r = j * 16 + l
                for m in range(F // 16):
                    outbuf[d, pl.ds(m * 16, 16)] = (
                        outbuf[d, pl.ds(m * 16, 16)]
                        + rows[r, pl.ds(m * 16, 16)])
            return gcarry

        lax.fori_loop(0, nb * 4, group, 0)

        @pl.when(cid <= 61)
        def _():
            pltpu.sync_copy(
                outbuf.at[pl.ds(0, CH)],
                out_hbm.at[pl.ds(pl.multiple_of(cid * CH, 8), CH)])

        @pl.when(cid == 62)
        def _():
            pltpu.sync_copy(outbuf.at[pl.ds(0, 80)],
                            out_hbm.at[pl.ds(9920, 80)])

        return carry

    lax.fori_loop(0, 2, chunk_body, 0)


@functools.cache
def _make_agg(F):
    mesh = plsc.VectorSubcoreMesh(core_axis_name="c", subcore_axis_name="s")
    return pl.kernel(
        functools.partial(_agg_body, F),
        out_type=jax.ShapeDtypeStruct((N, F), jnp.float32),
        mesh=mesh,
        scratch_types=[
            pltpu.VMEM((96,), jnp.int32),
            pltpu.VMEM((K,), jnp.int32),
            pltpu.VMEM((K,), jnp.int32),
            pltpu.VMEM((K,), jnp.int32),
            pltpu.VMEM((K, F), jnp.float32),
            pltpu.VMEM((OUT_ROWS, F), jnp.float32),
            pltpu.SemaphoreType.DMA,
        ],
    )


def _prep(edge_index):
    src, dst = edge_index[0], edge_index[1]
    e = src.shape[0]
    order = jnp.argsort(dst)
    src_s = jnp.take(src, order)
    dst_s = jnp.take(dst, order)
    dloc = (dst_s - (dst_s // CH) * CH).astype(jnp.int32)
    bounds = jnp.searchsorted(dst_s, jnp.arange(NCH + 1, dtype=jnp.int32) * CH)
    off = jnp.concatenate(
        [bounds.astype(jnp.int32), jnp.full((96 - NCH - 1,), e, jnp.int32)])
    rowptr = jnp.searchsorted(dst_s, jnp.arange(N + 1, dtype=jnp.int32))
    cnt = (rowptr[1:] - rowptr[:-1]).astype(jnp.float32)
    inv = (1.0 / jnp.maximum(cnt, 1.0)).reshape(N, 1)
    pad = jnp.zeros((E_EXTRA,), dtype=jnp.int32)
    return (jnp.concatenate([src_s.astype(jnp.int32), pad]),
            jnp.concatenate([dloc, pad]), off, inv)


def _sage_mm_body(agg_ref, inv_ref, h_ref, wl_ref, wr_ref, b_ref, out_ref):
    mean = agg_ref[...] * inv_ref[...]
    yl = lax.dot_general(mean, wl_ref[...], (((1,), (1,)), ((), ())),
                         preferred_element_type=jnp.float32)
    yr = lax.dot_general(h_ref[...], wr_ref[...], (((1,), (1,)), ((), ())),
                         preferred_element_type=jnp.float32)
    out_ref[...] = jnp.maximum(yl + yr + b_ref[...], 0.0)


def _sage_mm(agg, inv, h, wl, wr, b):
    f_in = h.shape[1]
    grid = N // M_BLK
    return pl.pallas_call(
        _sage_mm_body,
        grid=(grid,),
        in_specs=[
            pl.BlockSpec((M_BLK, f_in), lambda i: (i, 0)),
            pl.BlockSpec((M_BLK, 1), lambda i: (i, 0)),
            pl.BlockSpec((M_BLK, f_in), lambda i: (i, 0)),
            pl.BlockSpec((H, f_in), lambda i: (0, 0)),
            pl.BlockSpec((H, f_in), lambda i: (0, 0)),
            pl.BlockSpec((1, H), lambda i: (0, 0)),
        ],
        out_specs=pl.BlockSpec((M_BLK, H), lambda i: (i, 0)),
        out_shape=jax.ShapeDtypeStruct((N, H), jnp.float32),
    )(agg, inv, h, wl, wr, b.reshape(1, H))


def _layer(h, srcp, dstp, off, zeros, inv, wl, wr, b):
    agg = _make_agg(h.shape[1])(h, srcp, dstp, off, zeros)
    return _sage_mm(agg, inv, h, wl, wr, b)


def kernel(x, edge_index_connections, edge_index_destinations,
           W1l, W1r, b1, W2l, W2r, b2, W3l, W3r, b3, W4l, W4r, b4):
    src_c, dst_c, off_c, inv_c = _prep(edge_index_connections)
    src_d, dst_d, off_d, inv_d = _prep(edge_index_destinations)

    z256 = jnp.zeros((CH, 256), jnp.float32)
    z512 = jnp.zeros((CH, 512), jnp.float32)

    h = _layer(x, src_c, dst_c, off_c, z256, inv_c, W1l, W1r, b1)
    h = _layer(h, src_c, dst_c, off_c, z512, inv_c, W4l, W4r, b4)
    h = _layer(h, src_c, dst_c, off_c, z512, inv_c, W4l, W4r, b4)
    h = _layer(h, src_d, dst_d, off_d, z512, inv_d, W2l, W2r, b2)
    h = _layer(h, src_c, dst_c, off_c, z512, inv_c, W3l, W3r, b3)
    h = _layer(h, src_c, dst_c, off_c, z512, inv_c, W3l, W3r, b3)
    return h
